# baseline (speedup 1.0000x reference)
import functools

import jax
import jax.numpy as jnp
from jax import lax
from jax.experimental import pallas as pl
from jax.experimental.pallas import tpu as pltpu
from jax.experimental.pallas import tpu_sc as plsc

_ZERO_TOL = 1e-06
_K = 3


def _sc_wos(x, mbc, wbc, bbc, *, rows_sc):
    B, C, H, W = x.shape
    NC = mbc.shape[0]
    D2 = mbc.shape[1] // 16
    oh, ow = H - _K + 1, W - _K + 1
    n_sc = B * rows_sc * ow
    wpb = 32 // B
    rpw = rows_sc // wpb
    tpw = rpw * ow
    gpr = ow // 16
    ngroups = rpw * gpr
    half = D2 // 2

    mesh = plsc.VectorSubcoreMesh(core_axis_name="c", subcore_axis_name="s")

    @functools.partial(
        pl.kernel,
        mesh=mesh,
        out_type=jax.ShapeDtypeStruct((NC, n_sc), jnp.float32),
        scratch_types=[
            pltpu.VMEM((C, H, W), jnp.float32),
            pltpu.VMEM((NC, D2 * 16), jnp.float32),
            pltpu.VMEM((NC, D2 * 16), jnp.float32),
            pltpu.VMEM((NC, 16), jnp.float32),
            pltpu.VMEM((D2, 16), jnp.float32),
            pltpu.VMEM((NC, tpw), jnp.float32),
        ],
    )
    def k(x_hbm, m_hbm, w_hbm, b_hbm, out_hbm, xs, mb, wb, bb, vv, yb):
        cid = lax.axis_index("c")
        sid = lax.axis_index("s")
        wid = cid * 16 + sid
        bidx = wid // wpb
        r_off = (wid % wpb) * rpw

        pltpu.sync_copy(x_hbm.at[bidx], xs)
        pltpu.sync_copy(m_hbm, mb)
        pltpu.sync_copy(w_hbm, wb)
        pltpu.sync_copy(b_hbm, bb)

        big = jnp.float32(3.0e38)

        def group_body(g, _):
            gi = g // gpr
            gj = (g % gpr) * 16
            row0 = r_off + gi

            def chan_body(c, _):
                for ci in range(C):
                    for di in range(_K):
                        for dj in range(_K):
                            idx = ci * (_K * _K) + di * _K + dj
                            e = xs[ci, row0 + di, pl.ds(gj + dj, 16)]
                            vv[idx, :] = e + mb[c, pl.ds(idx * 16, 16)]
                            vv[idx + half, :] = (
                                mb[c, pl.ds((idx + half) * 16, 16)] - e)
                bvec = bb[c, :]

                def cand_body(j, carry):
                    ymin, ymax = carry
                    vj = vv[j, :]
                    wmj = wb[c, pl.ds(j * 16, 16)]
                    nzj = wmj > 0.0

                    def blk_body(t, acc):
                        jp0 = t * 6
                        terms = []
                        for kk in range(6):
                            jp = jp0 + kk
                            s = jnp.where(jp <= j, jnp.float32(1.0),
                                          jnp.float32(0.0))
                            vp = vv[jp, :]
                            wp = wb[c, pl.ds(jp * 16, 16)]
                            terms.append(jnp.where(vp > vj, wp, 0.0)
                                         + s * jnp.where(vp == vj, wp, 0.0))
                        t01 = terms[0] + terms[1]
                        t23 = terms[2] + terms[3]
                        t45 = terms[4] + terms[5]
                        return acc + (t01 + t23 + t45)

                    acc = lax.fori_loop(0, D2 // 6, blk_body,
                                        jnp.zeros((16,), jnp.float32))
                    ok = jnp.logical_and(acc <= bvec, nzj)
                    ymin = jnp.where(ok, jnp.minimum(ymin, vj), ymin)
                    ymax = jnp.where(nzj, jnp.maximum(ymax, vj), ymax)
                    return ymin, ymax

                init = (jnp.full((16,), big, jnp.float32),
                        jnp.full((16,), -big, jnp.float32))
                ymin, ymax = lax.fori_loop(0, D2, cand_body, init)
                y = jnp.where(ymin < big, ymin,
                              jnp.where(ymax > -big, ymax, jnp.float32(0.0)))
                yb[c, pl.ds(g * 16, 16)] = y
                return 0

            lax.fori_loop(0, NC, chan_body, 0)
            return 0

        lax.fori_loop(0, ngroups, group_body, 0)

        def wr_body(c, _):
            pltpu.sync_copy(yb.at[c],
                            out_hbm.at[c, pl.ds(wid * tpw, tpw)])
            return 0
        lax.fori_loop(0, NC, wr_body, 0)

    return k(x, mbc, wbc, bbc)


def _wos_body(x_ref, m_ref, w_ref, b_ref, out_ref, *, rows, ow, c_in, d2,
              row_lo=0):
    rc = pl.program_id(1)
    r0 = row_lo + rc * rows

    pieces = []
    for ci in range(c_in):
        xc = x_ref[0, ci, pl.ds(r0, rows + _K - 1), :]
        for di in range(_K):
            for dj in range(_K):
                p = xc[di:di + rows, dj:dj + ow]
                pieces.append(p[None])
    em = jnp.concatenate(pieces, axis=0)
    em = jnp.concatenate([em, -em], axis=0)
    em = jnp.concatenate([em, em], axis=2)

    def lane_pair(r):
        a = jnp.broadcast_to(r[0][:, :, None], (d2, 1, ow))
        b = jnp.broadcast_to(r[1][:, :, None], (d2, 1, ow))
        return jnp.concatenate([a, b], axis=2)

    mlane = lane_pair(m_ref[...])
    v = em + mlane

    wlane = lane_pair(w_ref[...])
    nzl = wlane > _ZERO_TOL
    wm = jnp.where(nzl, wlane, 0.0)

    br = b_ref[...]
    blane = jnp.concatenate(
        [jnp.broadcast_to(br[0], (1, ow)),
         jnp.broadcast_to(br[1], (1, ow))], axis=1)

    big = jnp.float32(3.0e38)
    ymin = jnp.full((rows, 2 * ow), big, jnp.float32)
    ymax = jnp.full((rows, 2 * ow), -big, jnp.float32)
    found = jnp.zeros((rows, 2 * ow), jnp.bool_)

    for j in range(d2):
        vj = v[j]
        lo = jnp.where(v[:j + 1] >= vj[None], wm[:j + 1], 0.0)
        cj = jnp.sum(lo, axis=0)
        if j + 1 < d2:
            hi = jnp.where(v[j + 1:] > vj[None], wm[j + 1:], 0.0)
            cj = cj + jnp.sum(hi, axis=0)
        nzj = nzl[j]
        ok = jnp.logical_and(cj <= blane, nzj)
        ymin = jnp.where(ok, jnp.minimum(ymin, vj), ymin)
        found = jnp.logical_or(found, ok)
        ymax = jnp.where(nzj, jnp.maximum(ymax, vj), ymax)

    y = jnp.where(found, ymin, jnp.where(ymax > -big, ymax, 0.0))
    out_ref[0] = y[:, :ow]
    out_ref[1] = y[:, ow:]


def _tc_wos(x, mask, weight, bias, *, row_lo):
    B, C, H, W = x.shape
    NC, D2 = weight.shape
    oh, ow = H - _K + 1, W - _K + 1
    rows = oh - row_lo
    RC = 1

    mask3 = mask.reshape(NC, D2, 1)
    weight3 = weight.reshape(NC, D2, 1)
    bias3 = bias.reshape(NC, 1, 1)

    body = functools.partial(_wos_body, rows=rows, ow=ow, c_in=C, d2=D2,
                             row_lo=row_lo)
    out3 = pl.pallas_call(
        body,
        grid=(B, RC, NC // 2),
        in_specs=[
            pl.BlockSpec((1, C, H, W), lambda b, rc, c: (b, 0, 0, 0)),
            pl.BlockSpec((2, D2, 1), lambda b, rc, c: (c, 0, 0)),
            pl.BlockSpec((2, D2, 1), lambda b, rc, c: (c, 0, 0)),
            pl.BlockSpec((2, 1, 1), lambda b, rc, c: (c, 0, 0)),
        ],
        out_specs=pl.BlockSpec(
            (2, rows, ow),
            lambda b, rc, c: (c, b * RC + rc, 0)),
        out_shape=jax.ShapeDtypeStruct((NC, B * rows, ow), jnp.float32),
    )(x, mask3, weight3, bias3)
    return out3


def kernel(x, mask, weight, bias):
    B, C, H, W = x.shape
    NC, D2 = weight.shape
    oh, ow = H - _K + 1, W - _K + 1
    N = B * oh * ow
    rows_sc = 64

    if rows_sc > 0:
        wm = jnp.where(weight > _ZERO_TOL, weight, 0.0)
        mbc = jnp.broadcast_to(
            mask[:, :, None], (NC, D2, 16)).reshape(NC, D2 * 16)
        wbc = jnp.broadcast_to(
            wm[:, :, None], (NC, D2, 16)).reshape(NC, D2 * 16)
        bbc = jnp.broadcast_to(bias.reshape(NC, 1), (NC, 16))
        y_sc = _sc_wos(x, mbc, wbc, bbc, rows_sc=rows_sc)

    if rows_sc == 0:
        yall = _tc_wos(x, mask, weight, bias, row_lo=0).reshape(NC, N)
    elif rows_sc < oh:
        y_tc = _tc_wos(x, mask, weight, bias, row_lo=rows_sc)
        ysc4 = y_sc.reshape(NC, B, rows_sc, ow)
        ytc4 = y_tc.reshape(NC, B, oh - rows_sc, ow)
        yall = jnp.concatenate([ysc4, ytc4], axis=2).reshape(NC, N)
    else:
        yall = y_sc
    return yall.T.reshape(-1, NC, oh, ow)


# speedup vs baseline: 7.2532x; 7.2532x over previous
import functools

import jax
import jax.numpy as jnp
from jax import lax
from jax.experimental import pallas as pl
from jax.experimental.pallas import tpu as pltpu
from jax.experimental.pallas import tpu_sc as plsc

_ZERO_TOL = 1e-06
_K = 3


def _sc_wos(x, mbc, wbc, bbc, *, rows_sc):
    B, C, H, W = x.shape
    NC = mbc.shape[0]
    D2 = mbc.shape[1] // 16
    oh, ow = H - _K + 1, W - _K + 1
    n_sc = B * rows_sc * ow
    wpb = 32 // B
    rpw = rows_sc // wpb
    tpw = rpw * ow
    gpr = ow // 16
    ngroups = rpw * gpr
    half = D2 // 2

    mesh = plsc.VectorSubcoreMesh(core_axis_name="c", subcore_axis_name="s")

    @functools.partial(
        pl.kernel,
        mesh=mesh,
        out_type=jax.ShapeDtypeStruct((NC, n_sc), jnp.float32),
        scratch_types=[
            pltpu.VMEM((C, H, W), jnp.float32),
            pltpu.VMEM((NC, D2 * 16), jnp.float32),
            pltpu.VMEM((NC, D2 * 16), jnp.float32),
            pltpu.VMEM((NC, 16), jnp.float32),
            pltpu.VMEM((D2, 16), jnp.float32),
            pltpu.VMEM((NC, tpw), jnp.float32),
        ],
    )
    def k(x_hbm, m_hbm, w_hbm, b_hbm, out_hbm, xs, mb, wb, bb, vv, yb):
        cid = lax.axis_index("c")
        sid = lax.axis_index("s")
        wid = cid * 16 + sid
        bidx = wid // wpb
        r_off = (wid % wpb) * rpw

        pltpu.sync_copy(x_hbm.at[bidx], xs)
        pltpu.sync_copy(m_hbm, mb)
        pltpu.sync_copy(w_hbm, wb)
        pltpu.sync_copy(b_hbm, bb)

        big = jnp.float32(3.0e38)

        def group_body(g, _):
            gi = g // gpr
            gj = (g % gpr) * 16
            row0 = r_off + gi

            def chan_body(c, _):
                for ci in range(C):
                    for di in range(_K):
                        for dj in range(_K):
                            idx = ci * (_K * _K) + di * _K + dj
                            e = xs[ci, row0 + di, pl.ds(gj + dj, 16)]
                            vv[idx, :] = e + mb[c, pl.ds(idx * 16, 16)]
                            vv[idx + half, :] = (
                                mb[c, pl.ds((idx + half) * 16, 16)] - e)
                bvec = bb[c, :]

                def cand_body(j, carry):
                    ymin, ymax = carry
                    vj = vv[j, :]
                    wmj = wb[c, pl.ds(j * 16, 16)]
                    nzj = wmj > 0.0

                    def blk_body(t, acc):
                        jp0 = t * 6
                        terms = []
                        for kk in range(6):
                            jp = jp0 + kk
                            s = jnp.where(jp <= j, jnp.float32(1.0),
                                          jnp.float32(0.0))
                            vp = vv[jp, :]
                            wp = wb[c, pl.ds(jp * 16, 16)]
                            terms.append(jnp.where(vp > vj, wp, 0.0)
                                         + s * jnp.where(vp == vj, wp, 0.0))
                        t01 = terms[0] + terms[1]
                        t23 = terms[2] + terms[3]
                        t45 = terms[4] + terms[5]
                        return acc + (t01 + t23 + t45)

                    acc = lax.fori_loop(0, D2 // 6, blk_body,
                                        jnp.zeros((16,), jnp.float32))
                    ok = jnp.logical_and(acc <= bvec, nzj)
                    ymin = jnp.where(ok, jnp.minimum(ymin, vj), ymin)
                    ymax = jnp.where(nzj, jnp.maximum(ymax, vj), ymax)
                    return ymin, ymax

                init = (jnp.full((16,), big, jnp.float32),
                        jnp.full((16,), -big, jnp.float32))
                ymin, ymax = lax.fori_loop(0, D2, cand_body, init)
                y = jnp.where(ymin < big, ymin,
                              jnp.where(ymax > -big, ymax, jnp.float32(0.0)))
                yb[c, pl.ds(g * 16, 16)] = y
                return 0

            lax.fori_loop(0, NC, chan_body, 0)
            return 0

        lax.fori_loop(0, ngroups, group_body, 0)

        def wr_body(c, _):
            pltpu.sync_copy(yb.at[c],
                            out_hbm.at[c, pl.ds(wid * tpw, tpw)])
            return 0
        lax.fori_loop(0, NC, wr_body, 0)

    return k(x, mbc, wbc, bbc)


def _wos_body(x_ref, m_ref, w_ref, b_ref, out_ref, *, rows, ow, c_in, d2,
              row_lo=0):
    rc = pl.program_id(1)
    r0 = row_lo + rc * rows

    pieces = []
    for ci in range(c_in):
        xc = x_ref[0, ci, pl.ds(r0, rows + _K - 1), :]
        for di in range(_K):
            for dj in range(_K):
                p = xc[di:di + rows, dj:dj + ow]
                pieces.append(p[None])
    em = jnp.concatenate(pieces, axis=0)
    em = jnp.concatenate([em, -em], axis=0)
    em = jnp.concatenate([em, em], axis=2)

    def lane_pair(r):
        a = jnp.broadcast_to(r[0][:, :, None], (d2, 1, ow))
        b = jnp.broadcast_to(r[1][:, :, None], (d2, 1, ow))
        return jnp.concatenate([a, b], axis=2)

    mlane = lane_pair(m_ref[...])
    v = em + mlane

    wlane = lane_pair(w_ref[...])
    nzl = wlane > _ZERO_TOL
    wm = jnp.where(nzl, wlane, 0.0)

    br = b_ref[...]
    blane = jnp.concatenate(
        [jnp.broadcast_to(br[0], (1, ow)),
         jnp.broadcast_to(br[1], (1, ow))], axis=1)

    big = jnp.float32(3.0e38)
    ymin = jnp.full((rows, 2 * ow), big, jnp.float32)
    ymax = jnp.full((rows, 2 * ow), -big, jnp.float32)
    found = jnp.zeros((rows, 2 * ow), jnp.bool_)

    for j in range(d2):
        vj = v[j]
        lo = jnp.where(v[:j + 1] >= vj[None], wm[:j + 1], 0.0)
        cj = jnp.sum(lo, axis=0)
        if j + 1 < d2:
            hi = jnp.where(v[j + 1:] > vj[None], wm[j + 1:], 0.0)
            cj = cj + jnp.sum(hi, axis=0)
        nzj = nzl[j]
        ok = jnp.logical_and(cj <= blane, nzj)
        ymin = jnp.where(ok, jnp.minimum(ymin, vj), ymin)
        found = jnp.logical_or(found, ok)
        ymax = jnp.where(nzj, jnp.maximum(ymax, vj), ymax)

    y = jnp.where(found, ymin, jnp.where(ymax > -big, ymax, 0.0))
    out_ref[0] = y[:, :ow]
    out_ref[1] = y[:, ow:]


def _tc_wos(x, mask, weight, bias, *, row_lo):
    B, C, H, W = x.shape
    NC, D2 = weight.shape
    oh, ow = H - _K + 1, W - _K + 1
    rows = oh - row_lo
    RC = 1

    mask3 = mask.reshape(NC, D2, 1)
    weight3 = weight.reshape(NC, D2, 1)
    bias3 = bias.reshape(NC, 1, 1)

    body = functools.partial(_wos_body, rows=rows, ow=ow, c_in=C, d2=D2,
                             row_lo=row_lo)
    out3 = pl.pallas_call(
        body,
        grid=(B, RC, NC // 2),
        in_specs=[
            pl.BlockSpec((1, C, H, W), lambda b, rc, c: (b, 0, 0, 0)),
            pl.BlockSpec((2, D2, 1), lambda b, rc, c: (c, 0, 0)),
            pl.BlockSpec((2, D2, 1), lambda b, rc, c: (c, 0, 0)),
            pl.BlockSpec((2, 1, 1), lambda b, rc, c: (c, 0, 0)),
        ],
        out_specs=pl.BlockSpec(
            (2, rows, ow),
            lambda b, rc, c: (c, b * RC + rc, 0)),
        out_shape=jax.ShapeDtypeStruct((NC, B * rows, ow), jnp.float32),
    )(x, mask3, weight3, bias3)
    return out3


def kernel(x, mask, weight, bias):
    B, C, H, W = x.shape
    NC, D2 = weight.shape
    oh, ow = H - _K + 1, W - _K + 1
    N = B * oh * ow
    rows_sc = 8

    if rows_sc > 0:
        wm = jnp.where(weight > _ZERO_TOL, weight, 0.0)
        mbc = jnp.broadcast_to(
            mask[:, :, None], (NC, D2, 16)).reshape(NC, D2 * 16)
        wbc = jnp.broadcast_to(
            wm[:, :, None], (NC, D2, 16)).reshape(NC, D2 * 16)
        bbc = jnp.broadcast_to(bias.reshape(NC, 1), (NC, 16))
        y_sc = _sc_wos(x, mbc, wbc, bbc, rows_sc=rows_sc)

    if rows_sc == 0:
        yall = _tc_wos(x, mask, weight, bias, row_lo=0).reshape(NC, N)
    elif rows_sc < oh:
        y_tc = _tc_wos(x, mask, weight, bias, row_lo=rows_sc)
        ysc4 = y_sc.reshape(NC, B, rows_sc, ow)
        ytc4 = y_tc.reshape(NC, B, oh - rows_sc, ow)
        yall = jnp.concatenate([ysc4, ytc4], axis=2).reshape(NC, N)
    else:
        yall = y_sc
    return yall.T.reshape(-1, NC, oh, ow)
